# Initial kernel scaffold; baseline (speedup 1.0000x reference)
import jax
import jax.numpy as jnp
from jax.experimental import pallas as pl


def kernel(x, edge_index, Wl, bl, Wr, br, gamma, beta, Wp1, bp1, Wp2, bp2):
    raise NotImplementedError("write your pallas kernel here")


# speedup vs baseline: 4.4379x; 4.4379x over previous
import functools

import jax
import jax.numpy as jnp
from jax import lax
from jax.experimental import pallas as pl
from jax.experimental.pallas import tpu as pltpu
from jax.experimental.pallas import tpu_sc as plsc

N = 10000
E = 320000
D = 128
OUT = 64

NC = 2
NS = 16
NW = NC * NS
EDGES_PER_W = E // NW
CHUNK = 80
NCHUNK = EDGES_PER_W // CHUNK
ZROWS = 8
ROWS_PER_S = 624


def _seg_sum_body(x_hbm, src_hbm, dst_hbm, out_hbm,
                  sidx, didx, rows, zbuf, acc, gsem):
    cid = lax.axis_index("c")
    sid = lax.axis_index("s")
    wid = cid * NS + sid

    z16 = jnp.zeros((16,), jnp.float32)
    for r in range(ZROWS):
        for j in range(D // 16):
            zbuf[r, pl.ds(j * 16, 16)] = z16

    row0 = sid * ROWS_PER_S
    nrows = jnp.where(sid == NS - 1, N - (NS - 1) * ROWS_PER_S, ROWS_PER_S)

    def zero_step(k, _):
        pltpu.sync_copy(zbuf, acc.at[pl.ds(row0 + k * ZROWS, ZROWS)])
        return 0

    lax.fori_loop(0, nrows // ZROWS, zero_step, 0)
    plsc.subcore_barrier()

    ebase = wid * EDGES_PER_W

    def edge_step(c, _):
        off = ebase + c * CHUNK
        pltpu.sync_copy(src_hbm.at[pl.ds(off, CHUNK)], sidx)
        pltpu.sync_copy(dst_hbm.at[pl.ds(off, CHUNK)], didx)
        pltpu.async_copy(x_hbm.at[sidx], rows, gsem).wait()
        pltpu.sync_copy(rows, acc.at[didx], add=True)
        return 0

    lax.fori_loop(0, NCHUNK, edge_step, 0)
    plsc.subcore_barrier()

    pltpu.sync_copy(acc.at[pl.ds(row0, nrows)],
                    out_hbm.at[cid, pl.ds(row0, nrows)])


@functools.cache
def _build_seg_sum():
    return pl.kernel(
        _seg_sum_body,
        mesh=plsc.VectorSubcoreMesh(core_axis_name="c", subcore_axis_name="s"),
        out_type=jax.ShapeDtypeStruct((NC, N, D), jnp.float32),
        scratch_types=[
            pltpu.VMEM((CHUNK,), jnp.int32),
            pltpu.VMEM((CHUNK,), jnp.int32),
            pltpu.VMEM((CHUNK, D), jnp.float32),
            pltpu.VMEM((ZROWS, D), jnp.float32),
            pltpu.VMEM_SHARED((N, D), jnp.float32),
            pltpu.SemaphoreType.DMA,
        ],
    )


def _seg_sum(x, src, dst):
    return _build_seg_sum()(x, src, dst)


def _layer_body(z0_ref, z1_ref, x_ref, wr_ref, br_ref, g_ref, b_ref,
                wl_ref, bl_ref, o_ref):
    z = z0_ref[...] + z1_ref[...]
    h = lax.dot_general(z, wr_ref[...], (((1,), (1,)), ((), ())),
                        preferred_element_type=jnp.float32) + br_ref[...]
    h = jnp.maximum(h, 0.0)
    mu = jnp.mean(h, axis=0, keepdims=True)
    var = jnp.mean((h - mu) ** 2, axis=0, keepdims=True)
    h = g_ref[...] * (h - mu) / jnp.sqrt(var + 1e-5) + b_ref[...]
    out = lax.dot_general(x_ref[...], wl_ref[...], (((1,), (1,)), ((), ())),
                          preferred_element_type=jnp.float32) + bl_ref[...]
    out = out + h + z
    nrm = jnp.sqrt(jnp.sum(out * out, axis=1, keepdims=True))
    out = out / jnp.maximum(nrm, 1e-12)
    o_ref[...] = jnp.maximum(out, 0.0)


def _final_body(z0_ref, z1_ref, x_ref, wr_ref, br_ref, g_ref, b_ref,
                wl_ref, bl_ref, wp1_ref, bp1_ref, wp2_ref, bp2_ref, o_ref):
    z = z0_ref[...] + z1_ref[...]
    h = lax.dot_general(z, wr_ref[...], (((1,), (1,)), ((), ())),
                        preferred_element_type=jnp.float32) + br_ref[...]
    h = jnp.maximum(h, 0.0)
    mu = jnp.mean(h, axis=0, keepdims=True)
    var = jnp.mean((h - mu) ** 2, axis=0, keepdims=True)
    h = g_ref[...] * (h - mu) / jnp.sqrt(var + 1e-5) + b_ref[...]
    out = lax.dot_general(x_ref[...], wl_ref[...], (((1,), (1,)), ((), ())),
                          preferred_element_type=jnp.float32) + bl_ref[...]
    out = out + h + z
    nrm = jnp.sqrt(jnp.sum(out * out, axis=1, keepdims=True))
    out = out / jnp.maximum(nrm, 1e-12)
    xo = jnp.maximum(out, 0.0)
    p = lax.dot_general(xo, wp1_ref[...], (((1,), (1,)), ((), ())),
                        preferred_element_type=jnp.float32) + bp1_ref[...]
    q = lax.dot_general(p, wp2_ref[...], (((1,), (1,)), ((), ())),
                        preferred_element_type=jnp.float32) + bp2_ref[...]
    m = jnp.max(q, axis=1, keepdims=True)
    s = q - m
    lse = jnp.log(jnp.sum(jnp.exp(s), axis=1, keepdims=True))
    o_ref[...] = s - lse


def _tc_layer(z01, x, wr, br, g, b, wl, bl):
    return pl.pallas_call(
        _layer_body,
        out_shape=jax.ShapeDtypeStruct((N, D), jnp.float32),
    )(z01[0], z01[1], x, wr, br, g, b, wl, bl)


def _tc_final(z01, x, wr, br, g, b, wl, bl, wp1, bp1, wp2, bp2):
    return pl.pallas_call(
        _final_body,
        out_shape=jax.ShapeDtypeStruct((N, OUT), jnp.float32),
    )(z01[0], z01[1], x, wr, br, g, b, wl, bl, wp1, bp1, wp2, bp2)


def kernel(x, edge_index, Wl, bl, Wr, br, gamma, beta, Wp1, bp1, Wp2, bp2):
    src = edge_index[0]
    dst = edge_index[1]
    for i in range(2):
        z01 = _seg_sum(x, src, dst)
        x = _tc_layer(z01, x, Wr[i], br[i], gamma[i], beta[i], Wl[i], bl[i])
    z01 = _seg_sum(x, src, dst)
    return _tc_final(z01, x, Wr[2], br[2], gamma[2], beta[2], Wl[2], bl[2],
                     Wp1, bp1, Wp2, bp2)


# speedup vs baseline: 9.9095x; 2.2329x over previous
import functools

import jax
import jax.numpy as jnp
from jax import lax
from jax.experimental import pallas as pl
from jax.experimental.pallas import tpu as pltpu
from jax.experimental.pallas import tpu_sc as plsc

N = 10000
E = 320000
D = 128
OUT = 64

NC = 2
NS = 16
NW = NC * NS
EDGES_PER_W = E // NW
CHUNK = 80
NCHUNK = EDGES_PER_W // CHUNK
RING = 2
ZROWS = 8
ROWS_PER_S = 624


ISLOTS = 4


def _seg_sum_body(x_hbm, src_hbm, dst_hbm, out_hbm,
                  didx, sbufs, dbuf, rows, zbuf, acc, gsems, ssems, isems):
    cid = lax.axis_index("c")
    sid = lax.axis_index("s")
    wid = cid * NS + sid
    ebase = wid * EDGES_PER_W

    z16 = jnp.zeros((16,), jnp.float32)
    for r in range(ZROWS):
        for j in range(D // 16):
            zbuf[r, pl.ds(j * 16, 16)] = z16

    row0 = sid * ROWS_PER_S
    nrows = jnp.where(sid == NS - 1, N - (NS - 1) * ROWS_PER_S, ROWS_PER_S)

    def zero_step(k, _):
        pltpu.sync_copy(zbuf, acc.at[pl.ds(row0 + k * ZROWS, ZROWS)])
        return 0

    lax.fori_loop(0, nrows // ZROWS, zero_step, 0)

    pltpu.sync_copy(dst_hbm.at[pl.ds(ebase, EDGES_PER_W)], didx)
    plsc.subcore_barrier()

    def copy_idx(c, dstref):
        for j in range(CHUNK // 16):
            dstref[pl.ds(j * 16, 16)] = didx[pl.ds(c * CHUNK + j * 16, 16)]

    def fetch_idx(c, u):
        pltpu.async_copy(src_hbm.at[pl.ds(ebase + c * CHUNK, CHUNK)],
                         sbufs[u], isems[u])

    def wait_idx(u):
        pltpu.make_async_copy(src_hbm.at[pl.ds(0, CHUNK)], sbufs[u],
                              isems[u]).wait()

    def wait_gather(b):
        pltpu.make_async_copy(x_hbm.at[sbufs[0]], rows.at[b],
                              gsems[b]).wait()

    def scatter(c, b):
        copy_idx(c, dbuf)
        pltpu.async_copy(rows.at[b], acc.at[dbuf], ssems[b],
                         add=True).wait()

    for u in range(ISLOTS):
        fetch_idx(u, u)
    for b in range(RING):
        wait_idx(b)
        pltpu.async_copy(x_hbm.at[sbufs[b]], rows.at[b], gsems[b])

    def edge_step(k2, _):
        c0 = k2 * ISLOTS
        for u in range(ISLOTS):
            b = u % RING
            u2 = (u + 2) % ISLOTS
            wait_gather(b)
            scatter(c0 + u, b)
            wait_idx(u2)
            pltpu.async_copy(x_hbm.at[sbufs[u2]], rows.at[b], gsems[b])
            fetch_idx(c0 + u + ISLOTS, u)
        return 0

    lax.fori_loop(0, (NCHUNK - 5) // ISLOTS, edge_step, 0)

    t = NCHUNK - 5
    wait_gather(0)
    scatter(t, 0)
    wait_idx(2)
    pltpu.async_copy(x_hbm.at[sbufs[2]], rows.at[0], gsems[0])
    fetch_idx(t + 4, 0)
    wait_gather(1)
    scatter(t + 1, 1)
    wait_idx(3)
    pltpu.async_copy(x_hbm.at[sbufs[3]], rows.at[1], gsems[1])
    wait_gather(0)
    scatter(t + 2, 0)
    wait_idx(0)
    pltpu.async_copy(x_hbm.at[sbufs[0]], rows.at[0], gsems[0])
    wait_gather(1)
    scatter(t + 3, 1)
    wait_gather(0)
    scatter(t + 4, 0)
    plsc.subcore_barrier()

    pltpu.sync_copy(acc.at[pl.ds(row0, nrows)],
                    out_hbm.at[cid, pl.ds(row0, nrows)])


@functools.cache
def _build_seg_sum():
    return pl.kernel(
        _seg_sum_body,
        mesh=plsc.VectorSubcoreMesh(core_axis_name="c", subcore_axis_name="s"),
        out_type=jax.ShapeDtypeStruct((NC, N, D), jnp.float32),
        scratch_types=[
            pltpu.VMEM((EDGES_PER_W,), jnp.int32),
            [pltpu.VMEM((CHUNK,), jnp.int32)] * ISLOTS,
            pltpu.VMEM((CHUNK,), jnp.int32),
            pltpu.VMEM((RING, CHUNK, D), jnp.float32),
            pltpu.VMEM((ZROWS, D), jnp.float32),
            pltpu.VMEM_SHARED((N, D), jnp.float32),
            [pltpu.SemaphoreType.DMA] * RING,
            [pltpu.SemaphoreType.DMA] * RING,
            [pltpu.SemaphoreType.DMA] * ISLOTS,
        ],
    )


def _seg_sum(x, src, dst):
    return _build_seg_sum()(x, src, dst)


def _layer_body(z0_ref, z1_ref, x_ref, wr_ref, br_ref, g_ref, b_ref,
                wl_ref, bl_ref, o_ref):
    z = z0_ref[...] + z1_ref[...]
    h = lax.dot_general(z, wr_ref[...], (((1,), (1,)), ((), ())),
                        preferred_element_type=jnp.float32) + br_ref[...]
    h = jnp.maximum(h, 0.0)
    mu = jnp.mean(h, axis=0, keepdims=True)
    var = jnp.mean((h - mu) ** 2, axis=0, keepdims=True)
    h = g_ref[...] * (h - mu) / jnp.sqrt(var + 1e-5) + b_ref[...]
    out = lax.dot_general(x_ref[...], wl_ref[...], (((1,), (1,)), ((), ())),
                          preferred_element_type=jnp.float32) + bl_ref[...]
    out = out + h + z
    nrm = jnp.sqrt(jnp.sum(out * out, axis=1, keepdims=True))
    out = out / jnp.maximum(nrm, 1e-12)
    o_ref[...] = jnp.maximum(out, 0.0)


def _final_body(z0_ref, z1_ref, x_ref, wr_ref, br_ref, g_ref, b_ref,
                wl_ref, bl_ref, wp1_ref, bp1_ref, wp2_ref, bp2_ref, o_ref):
    z = z0_ref[...] + z1_ref[...]
    h = lax.dot_general(z, wr_ref[...], (((1,), (1,)), ((), ())),
                        preferred_element_type=jnp.float32) + br_ref[...]
    h = jnp.maximum(h, 0.0)
    mu = jnp.mean(h, axis=0, keepdims=True)
    var = jnp.mean((h - mu) ** 2, axis=0, keepdims=True)
    h = g_ref[...] * (h - mu) / jnp.sqrt(var + 1e-5) + b_ref[...]
    out = lax.dot_general(x_ref[...], wl_ref[...], (((1,), (1,)), ((), ())),
                          preferred_element_type=jnp.float32) + bl_ref[...]
    out = out + h + z
    nrm = jnp.sqrt(jnp.sum(out * out, axis=1, keepdims=True))
    out = out / jnp.maximum(nrm, 1e-12)
    xo = jnp.maximum(out, 0.0)
    p = lax.dot_general(xo, wp1_ref[...], (((1,), (1,)), ((), ())),
                        preferred_element_type=jnp.float32) + bp1_ref[...]
    q = lax.dot_general(p, wp2_ref[...], (((1,), (1,)), ((), ())),
                        preferred_element_type=jnp.float32) + bp2_ref[...]
    m = jnp.max(q, axis=1, keepdims=True)
    s = q - m
    lse = jnp.log(jnp.sum(jnp.exp(s), axis=1, keepdims=True))
    o_ref[...] = s - lse


def _tc_layer(z01, x, wr, br, g, b, wl, bl):
    return pl.pallas_call(
        _layer_body,
        out_shape=jax.ShapeDtypeStruct((N, D), jnp.float32),
    )(z01[0], z01[1], x, wr, br, g, b, wl, bl)


def _tc_final(z01, x, wr, br, g, b, wl, bl, wp1, bp1, wp2, bp2):
    return pl.pallas_call(
        _final_body,
        out_shape=jax.ShapeDtypeStruct((N, OUT), jnp.float32),
    )(z01[0], z01[1], x, wr, br, g, b, wl, bl, wp1, bp1, wp2, bp2)


def kernel(x, edge_index, Wl, bl, Wr, br, gamma, beta, Wp1, bp1, Wp2, bp2):
    src = edge_index[0]
    dst = edge_index[1]
    for i in range(2):
        z01 = _seg_sum(x, src, dst)
        x = _tc_layer(z01, x, Wr[i], br[i], gamma[i], beta[i], Wl[i], bl[i])
    z01 = _seg_sum(x, src, dst)
    return _tc_final(z01, x, Wr[2], br[2], gamma[2], beta[2], Wl[2], bl[2],
                     Wp1, bp1, Wp2, bp2)
